# Initial kernel scaffold; baseline (speedup 1.0000x reference)
import jax
import jax.numpy as jnp
from jax.experimental import pallas as pl


def kernel(x, tables, W0, W1, W2):
    raise NotImplementedError("write your pallas kernel here")


# speedup vs baseline: 43.4600x; 43.4600x over previous
import functools

import jax
import jax.numpy as jnp
from jax import lax
from jax.experimental import pallas as pl
from jax.experimental.pallas import tpu as pltpu
from jax.experimental.pallas import tpu_sc as plsc

N_LEVELS = 12
F = 2
LOG2_T = 19
T = 1 << LOG2_T
BASE_RES = 16
SCALE = 1.3819
RES = [int((BASE_RES * SCALE**l) // 1) for l in range(N_LEVELS)]
DENSE = [(r + 1) ** 3 <= T for r in RES]
P1 = 2654435761
P2 = 805459861
CORNERS = [(i, j, k) for i in (0, 1) for j in (0, 1) for k in (0, 1)]

NW = 32
B = 512
GROUPS = B // 16
NSEG = (16 * B) // 128


def _encode(xx, xy, xz, tw):
    N = xx.shape[0]
    npt = N // NW
    nch = npt // B
    mesh = plsc.VectorSubcoreMesh(core_axis_name="c", subcore_axis_name="s")

    @functools.partial(
        pl.kernel,
        out_type=jax.ShapeDtypeStruct((2 * N_LEVELS, N), jnp.float32),
        mesh=mesh,
        scratch_types=[
            pltpu.VMEM((B,), jnp.float32),
            pltpu.VMEM((B,), jnp.float32),
            pltpu.VMEM((B,), jnp.float32),
            pltpu.VMEM((2 * N_LEVELS, B), jnp.float32),
            pltpu.VMEM((16 * B,), jnp.int32),
            pltpu.VMEM((16 * B,), jnp.float32),
            pltpu.SemaphoreType.DMA,
        ],
    )
    def enc(xx_h, xy_h, xz_h, tw_h, pe_h, xbx, xby, xbz, peb, idxb, dstb, sem):
        wid = lax.axis_index("s") * 2 + lax.axis_index("c")

        def chunk_body(ch, carry):
            base = wid * npt + ch * B
            pltpu.sync_copy(xx_h.at[pl.ds(base, B)], xbx)
            pltpu.sync_copy(xy_h.at[pl.ds(base, B)], xby)
            pltpu.sync_copy(xz_h.at[pl.ds(base, B)], xbz)

            for l in range(N_LEVELS):
                res = RES[l]
                resf = float(res)
                lbase = l * T * 2

                def idx_body(g, c, l=l, res=res, resf=resf, lbase=lbase):
                    s = pl.ds(g * 16, 16)
                    ix = jnp.clip((xbx[s] * resf).astype(jnp.int32), 0, res - 1)
                    iy = jnp.clip((xby[s] * resf).astype(jnp.int32), 0, res - 1)
                    iz = jnp.clip((xbz[s] * resf).astype(jnp.int32), 0, res - 1)
                    if DENSE[l]:
                        stride = res + 1
                        ax = (ix, ix + 1)
                        ay = (iy * stride, iy * stride + stride)
                        az = (iz * (stride * stride),
                              iz * (stride * stride) + stride * stride)
                        rows = [ax[i] + ay[j] + az[k] for (i, j, k) in CORNERS]
                    else:
                        ux, uy, uz = (ix.astype(jnp.uint32), iy.astype(jnp.uint32),
                                      iz.astype(jnp.uint32))
                        hx = (ux, ux + jnp.uint32(1))
                        hy = (uy * jnp.uint32(P1), uy * jnp.uint32(P1) + jnp.uint32(P1))
                        hz = (uz * jnp.uint32(P2), uz * jnp.uint32(P2) + jnp.uint32(P2))
                        rows = [((hx[i] ^ hy[j] ^ hz[k]) & jnp.uint32(T - 1)).astype(jnp.int32)
                                for (i, j, k) in CORNERS]
                    for ci in range(8):
                        w0 = rows[ci] * 2 + lbase
                        idxb[pl.ds((2 * ci) * B + g * 16, 16)] = w0
                        idxb[pl.ds((2 * ci + 1) * B + g * 16, 16)] = w0 + 1
                    return c

                lax.fori_loop(0, GROUPS, idx_body, 0)
                pltpu.async_copy(tw_h.at[idxb], dstb, sem).wait()

                def acc_body(g, c, l=l, resf=resf, res=res):
                    s = pl.ds(g * 16, 16)
                    fx = xbx[s] * resf
                    fy = xby[s] * resf
                    fz = xbz[s] * resf
                    wx = fx - jnp.clip(fx.astype(jnp.int32), 0, res - 1).astype(jnp.float32)
                    wy = fy - jnp.clip(fy.astype(jnp.int32), 0, res - 1).astype(jnp.float32)
                    wz = fz - jnp.clip(fz.astype(jnp.int32), 0, res - 1).astype(jnp.float32)
                    u = (1.0 - wx, wx)
                    v = (1.0 - wy, wy)
                    t = (1.0 - wz, wz)
                    pxy = [u[i] * v[j] for i in (0, 1) for j in (0, 1)]
                    acc0 = None
                    acc1 = None
                    for ci, (i, j, k) in enumerate(CORNERS):
                        wc = pxy[2 * i + j] * t[k]
                        v0 = dstb[pl.ds((2 * ci) * B + g * 16, 16)]
                        v1 = dstb[pl.ds((2 * ci + 1) * B + g * 16, 16)]
                        if acc0 is None:
                            acc0 = wc * v0
                            acc1 = wc * v1
                        else:
                            acc0 = acc0 + wc * v0
                            acc1 = acc1 + wc * v1
                    peb[2 * l, pl.ds(g * 16, 16)] = acc0
                    peb[2 * l + 1, pl.ds(g * 16, 16)] = acc1
                    return c

                lax.fori_loop(0, GROUPS, acc_body, 0)

            pltpu.sync_copy(peb, pe_h.at[:, pl.ds(base, B)])
            return carry

        lax.fori_loop(0, nch, chunk_body, 0)

    return enc(xx, xy, xz, tw).T


def _mlp(pe, W0, W1, W2):
    N = pe.shape[0]
    BN = 1024
    dn = (((1,), (0,)), ((), ()))

    def body(pe_ref, w0_ref, w1_ref, w2_ref, z_ref, d_ref):
        p = pe_ref[...]
        h = jnp.maximum(
            lax.dot_general(p, w0_ref[...], dn, preferred_element_type=jnp.float32), 0.0)
        h = jnp.maximum(
            lax.dot_general(h, w1_ref[...], dn, preferred_element_type=jnp.float32), 0.0)
        z = lax.dot_general(h, w2_ref[...], dn, preferred_element_type=jnp.float32)
        z_ref[...] = z
        z0 = z[:, 0:1]
        d_ref[...] = jnp.maximum(z0, 0.0) + jnp.log1p(jnp.exp(-jnp.abs(z0)))

    z, dens = pl.pallas_call(
        body,
        grid=(N // BN,),
        in_specs=[
            pl.BlockSpec((BN, 2 * N_LEVELS), lambda i: (i, 0)),
            pl.BlockSpec((2 * N_LEVELS, 64), lambda i: (0, 0)),
            pl.BlockSpec((64, 64), lambda i: (0, 0)),
            pl.BlockSpec((64, 16), lambda i: (0, 0)),
        ],
        out_specs=[
            pl.BlockSpec((BN, 16), lambda i: (i, 0)),
            pl.BlockSpec((BN, 1), lambda i: (i, 0)),
        ],
        out_shape=[
            jax.ShapeDtypeStruct((N, 16), jnp.float32),
            jax.ShapeDtypeStruct((N, 1), jnp.float32),
        ],
    )(pe, W0, W1, W2)
    return z, dens


def kernel(x, tables, W0, W1, W2):
    N = x.shape[0]
    xn = x * jnp.float32(1.0 / 256.0)
    xnt = xn.T
    pe = _encode(xnt[0], xnt[1], xnt[2], tables.reshape(-1))
    z, dens = _mlp(pe, W0, W1, W2)
    return (dens.reshape(x.shape[:-1]), pe, z)


# speedup vs baseline: 46.5163x; 1.0703x over previous
import functools

import jax
import jax.numpy as jnp
from jax import lax
from jax.experimental import pallas as pl
from jax.experimental.pallas import tpu as pltpu
from jax.experimental.pallas import tpu_sc as plsc

N_LEVELS = 12
F = 2
LOG2_T = 19
T = 1 << LOG2_T
BASE_RES = 16
SCALE = 1.3819
RES = [int((BASE_RES * SCALE**l) // 1) for l in range(N_LEVELS)]
DENSE = [(r + 1) ** 3 <= T for r in RES]
P1 = 2654435761
P2 = 805459861
CORNERS = [(i, j, k) for i in (0, 1) for j in (0, 1) for k in (0, 1)]

NW = 32
B = 512
GROUPS = B // 16
N_RESIDENT = 2
TAB0_ROWS = -(-((RES[0] + 1) ** 3) // 8) * 8
TAB1_ROWS = -(-((RES[1] + 1) ** 3) // 8) * 8


def _encode(xx, xy, xz, tw2):
    N = xx.shape[0]
    npt = N // NW
    nch = npt // B
    mesh = plsc.VectorSubcoreMesh(core_axis_name="c", subcore_axis_name="s")

    @functools.partial(
        pl.kernel,
        out_type=jax.ShapeDtypeStruct((2 * N_LEVELS, N), jnp.float32),
        mesh=mesh,
        compiler_params=pltpu.CompilerParams(
            needs_layout_passes=False, use_tc_tiling_on_sc=False),
        scratch_types=[
            pltpu.VMEM((B,), jnp.float32),
            pltpu.VMEM((B,), jnp.float32),
            pltpu.VMEM((B,), jnp.float32),
            pltpu.VMEM((2 * N_LEVELS, B), jnp.float32),
            pltpu.VMEM((16 * B,), jnp.int32),
            pltpu.VMEM((16 * B,), jnp.int32),
            pltpu.VMEM((16 * B,), jnp.float32),
            pltpu.VMEM((16 * B,), jnp.float32),
            pltpu.VMEM((TAB0_ROWS * 2,), jnp.float32),
            pltpu.VMEM((TAB1_ROWS * 2,), jnp.float32),
            pltpu.SemaphoreType.DMA,
            pltpu.SemaphoreType.DMA,
        ],
    )
    def enc(xx_h, xy_h, xz_h, twf_h, pe_h,
            xbx, xby, xbz, peb, idxA, idxB, dstA, dstB, tab0, tab1,
            semA, semB):
        wid = lax.axis_index("s") * 2 + lax.axis_index("c")
        idxb = (idxA, idxB)
        dstb = (dstA, dstB)
        sems = (semA, semB)

        pltpu.sync_copy(twf_h.at[pl.ds(0, TAB0_ROWS * 2)], tab0)
        pltpu.sync_copy(twf_h.at[pl.ds(2 * T, TAB1_ROWS * 2)], tab1)

        iota16 = lax.iota(jnp.int32, 16)
        zeros16 = jnp.zeros((16,), jnp.int32)
        ones16 = jnp.full((16,), 1, jnp.int32)

        def coords(g, l):
            res = RES[l]
            resf = float(res)
            s = pl.ds(g * 16, 16)
            fx = xbx[s] * resf
            fy = xby[s] * resf
            fz = xbz[s] * resf
            ix = jnp.clip(fx.astype(jnp.int32), 0, res - 1)
            iy = jnp.clip(fy.astype(jnp.int32), 0, res - 1)
            iz = jnp.clip(fz.astype(jnp.int32), 0, res - 1)
            return fx, fy, fz, ix, iy, iz

        def corner_rows(l, ix, iy, iz):
            res = RES[l]
            if DENSE[l]:
                stride = res + 1
                ax = (ix, ix + 1)
                ay = (iy * stride, iy * stride + stride)
                az = (iz * (stride * stride), iz * (stride * stride) + stride * stride)
                return [ax[i] + ay[j] + az[k] for (i, j, k) in CORNERS]
            ux, uy, uz = (ix.astype(jnp.uint32), iy.astype(jnp.uint32),
                          iz.astype(jnp.uint32))
            hx = (ux, ux + jnp.uint32(1))
            hy = (uy * jnp.uint32(P1), uy * jnp.uint32(P1) + jnp.uint32(P1))
            hz = (uz * jnp.uint32(P2), uz * jnp.uint32(P2) + jnp.uint32(P2))
            return [((hx[i] ^ hy[j] ^ hz[k]) & jnp.uint32(T - 1)).astype(jnp.int32)
                    for (i, j, k) in CORNERS]

        def weights(fx, fy, fz, ix, iy, iz):
            wx = fx - ix.astype(jnp.float32)
            wy = fy - iy.astype(jnp.float32)
            wz = fz - iz.astype(jnp.float32)
            u = (1.0 - wx, wx)
            v = (1.0 - wy, wy)
            t = (1.0 - wz, wz)
            pxy = [u[i] * v[j] for i in (0, 1) for j in (0, 1)]
            return pxy, t

        def chunk_body(ch, carry):
            base = wid * npt + ch * B
            pltpu.sync_copy(xx_h.at[pl.ds(base, B)], xbx)
            pltpu.sync_copy(xy_h.at[pl.ds(base, B)], xby)
            pltpu.sync_copy(xz_h.at[pl.ds(base, B)], xbz)

            def idx_loop(l):
                lbase = 2 * l * T
                bi = l % 2

                def body(g, c, l=l, lbase=lbase, bi=bi):
                    _, _, _, ix, iy, iz = coords(g, l)
                    rows = corner_rows(l, ix, iy, iz)
                    for ci in range(8):
                        w0 = rows[ci] * 2 + lbase
                        idxb[bi][pl.ds((2 * ci) * B + g * 16, 16)] = w0
                        idxb[bi][pl.ds((2 * ci + 1) * B + g * 16, 16)] = w0 + 1
                    return c

                lax.fori_loop(0, GROUPS, body, 0)

            def fire(l):
                bi = l % 2
                return pltpu.async_copy(twf_h.at[idxb[bi]], dstb[bi], sems[bi])

            def acc_loop(l):
                bi = l % 2

                def body(g, c, l=l, bi=bi):
                    fx, fy, fz, ix, iy, iz = coords(g, l)
                    pxy, t = weights(fx, fy, fz, ix, iy, iz)
                    acc0 = acc1 = None
                    for ci, (i, j, k) in enumerate(CORNERS):
                        wc = pxy[2 * i + j] * t[k]
                        v0 = dstb[bi][pl.ds((2 * ci) * B + g * 16, 16)]
                        v1 = dstb[bi][pl.ds((2 * ci + 1) * B + g * 16, 16)]
                        if acc0 is None:
                            acc0, acc1 = wc * v0, wc * v1
                        else:
                            acc0, acc1 = acc0 + wc * v0, acc1 + wc * v1
                    peb[2 * l, pl.ds(g * 16, 16)] = acc0
                    peb[2 * l + 1, pl.ds(g * 16, 16)] = acc1
                    return c

                lax.fori_loop(0, GROUPS, body, 0)

            def resident_loop(l, tab):
                def body(g, c, l=l):
                    fx, fy, fz, ix, iy, iz = coords(g, l)
                    rows = corner_rows(l, ix, iy, iz)
                    pxy, t = weights(fx, fy, fz, ix, iy, iz)
                    acc0 = acc1 = None
                    for ci, (i, j, k) in enumerate(CORNERS):
                        wc = pxy[2 * i + j] * t[k]
                        wi = rows[ci] * 2
                        v0 = plsc.load_gather(tab, [wi])
                        v1 = plsc.load_gather(tab, [wi + 1])
                        if acc0 is None:
                            acc0, acc1 = wc * v0, wc * v1
                        else:
                            acc0, acc1 = acc0 + wc * v0, acc1 + wc * v1
                    peb[2 * l, pl.ds(g * 16, 16)] = acc0
                    peb[2 * l + 1, pl.ds(g * 16, 16)] = acc1
                    return c

                lax.fori_loop(0, GROUPS, body, 0)

            idx_loop(2)
            d_prev = fire(2)
            resident_loop(0, tab0)
            resident_loop(1, tab1)
            for l in range(3, N_LEVELS):
                idx_loop(l)
                d_next = fire(l)
                d_prev.wait()
                acc_loop(l - 1)
                d_prev = d_next
            d_prev.wait()
            acc_loop(N_LEVELS - 1)

            pltpu.sync_copy(peb, pe_h.at[:, pl.ds(base, B)])
            return carry

        lax.fori_loop(0, nch, chunk_body, 0)

    return enc(xx, xy, xz, tw2.reshape(-1))


def _mlp(pe, W0, W1, W2):
    N = pe.shape[0]
    BN = 1024
    dn = (((1,), (0,)), ((), ()))

    def body(pe_ref, w0_ref, w1_ref, w2_ref, z_ref, d_ref):
        p = pe_ref[...]
        h = jnp.maximum(
            lax.dot_general(p, w0_ref[...], dn, preferred_element_type=jnp.float32), 0.0)
        h = jnp.maximum(
            lax.dot_general(h, w1_ref[...], dn, preferred_element_type=jnp.float32), 0.0)
        z = lax.dot_general(h, w2_ref[...], dn, preferred_element_type=jnp.float32)
        z_ref[...] = z
        z0 = z[:, 0:1]
        d_ref[...] = jnp.maximum(z0, 0.0) + jnp.log1p(jnp.exp(-jnp.abs(z0)))

    z, dens = pl.pallas_call(
        body,
        grid=(N // BN,),
        in_specs=[
            pl.BlockSpec((BN, 2 * N_LEVELS), lambda i: (i, 0)),
            pl.BlockSpec((2 * N_LEVELS, 64), lambda i: (0, 0)),
            pl.BlockSpec((64, 64), lambda i: (0, 0)),
            pl.BlockSpec((64, 16), lambda i: (0, 0)),
        ],
        out_specs=[
            pl.BlockSpec((BN, 16), lambda i: (i, 0)),
            pl.BlockSpec((BN, 1), lambda i: (i, 0)),
        ],
        out_shape=[
            jax.ShapeDtypeStruct((N, 16), jnp.float32),
            jax.ShapeDtypeStruct((N, 1), jnp.float32),
        ],
    )(pe, W0, W1, W2)
    return z, dens


def kernel(x, tables, W0, W1, W2):
    N = x.shape[0]
    xn = x * jnp.float32(1.0 / 256.0)
    xnt = xn.T
    pe = _encode(xnt[0], xnt[1], xnt[2], tables.reshape(N_LEVELS * T, F)).T
    z, dens = _mlp(pe, W0, W1, W2)
    return (dens.reshape(x.shape[:-1]), pe, z)


# speedup vs baseline: 56.0081x; 1.2041x over previous
import functools

import jax
import jax.numpy as jnp
from jax import lax
from jax.experimental import pallas as pl
from jax.experimental.pallas import tpu as pltpu
from jax.experimental.pallas import tpu_sc as plsc

N_LEVELS = 12
F = 2
LOG2_T = 19
T = 1 << LOG2_T
BASE_RES = 16
SCALE = 1.3819
RES = [int((BASE_RES * SCALE**l) // 1) for l in range(N_LEVELS)]
DENSE = [(r + 1) ** 3 <= T for r in RES]
P1 = 2654435761
P2 = 805459861
CORNERS = [(i, j, k) for i in (0, 1) for j in (0, 1) for k in (0, 1)]

NW = 32
B = 512
GROUPS = B // 16
N_RESIDENT = 2
TAB0_ROWS = -(-((RES[0] + 1) ** 3) // 32) * 32
TAB1_ROWS = -(-((RES[1] + 1) ** 3) // 32) * 32


def _encode(xx, xy, xz, tw2):
    N = xx.shape[0]
    npt = N // NW
    nch = npt // B
    mesh = plsc.VectorSubcoreMesh(core_axis_name="c", subcore_axis_name="s")

    @functools.partial(
        pl.kernel,
        out_type=jax.ShapeDtypeStruct((2 * N_LEVELS, N), jnp.float32),
        mesh=mesh,
        compiler_params=pltpu.CompilerParams(
            needs_layout_passes=False, use_tc_tiling_on_sc=False),
        scratch_types=[
            pltpu.VMEM((B,), jnp.float32),
            pltpu.VMEM((B,), jnp.float32),
            pltpu.VMEM((B,), jnp.float32),
            pltpu.VMEM((2 * N_LEVELS, B), jnp.float32),
            pltpu.VMEM((8 * B,), jnp.int32),
            pltpu.VMEM((8 * B,), jnp.int32),
            pltpu.VMEM((8 * B, 8), jnp.float32),
            pltpu.VMEM((8 * B, 8), jnp.float32),
            pltpu.VMEM((TAB0_ROWS * 2 // 8, 8), jnp.float32),
            pltpu.VMEM((TAB1_ROWS * 2 // 8, 8), jnp.float32),
            pltpu.SemaphoreType.DMA,
            pltpu.SemaphoreType.DMA,
        ],
    )
    def enc(xx_h, xy_h, xz_h, tw8_h, pe_h,
            xbx, xby, xbz, peb, idxA, idxB, dstA, dstB, tab0, tab1,
            semA, semB):
        wid = lax.axis_index("s") * 2 + lax.axis_index("c")
        idxb = (idxA, idxB)
        dstb = (dstA, dstB)
        sems = (semA, semB)

        pltpu.sync_copy(tw8_h.at[pl.ds(0, TAB0_ROWS * 2 // 8)], tab0)
        pltpu.sync_copy(tw8_h.at[pl.ds(T // 4, TAB1_ROWS * 2 // 8)], tab1)

        iota16 = lax.iota(jnp.int32, 16)
        zeros16 = jnp.zeros((16,), jnp.int32)
        ones16 = jnp.full((16,), 1, jnp.int32)

        def coords(g, l):
            res = RES[l]
            resf = float(res)
            s = pl.ds(g * 16, 16)
            fx = xbx[s] * resf
            fy = xby[s] * resf
            fz = xbz[s] * resf
            ix = jnp.clip(fx.astype(jnp.int32), 0, res - 1)
            iy = jnp.clip(fy.astype(jnp.int32), 0, res - 1)
            iz = jnp.clip(fz.astype(jnp.int32), 0, res - 1)
            return fx, fy, fz, ix, iy, iz

        def corner_rows(l, ix, iy, iz):
            res = RES[l]
            if DENSE[l]:
                stride = res + 1
                ax = (ix, ix + 1)
                ay = (iy * stride, iy * stride + stride)
                az = (iz * (stride * stride), iz * (stride * stride) + stride * stride)
                return [ax[i] + ay[j] + az[k] for (i, j, k) in CORNERS]
            ux, uy, uz = (ix.astype(jnp.uint32), iy.astype(jnp.uint32),
                          iz.astype(jnp.uint32))
            hx = (ux, ux + jnp.uint32(1))
            hy = (uy * jnp.uint32(P1), uy * jnp.uint32(P1) + jnp.uint32(P1))
            hz = (uz * jnp.uint32(P2), uz * jnp.uint32(P2) + jnp.uint32(P2))
            return [((hx[i] ^ hy[j] ^ hz[k]) & jnp.uint32(T - 1)).astype(jnp.int32)
                    for (i, j, k) in CORNERS]

        def weights(fx, fy, fz, ix, iy, iz):
            wx = fx - ix.astype(jnp.float32)
            wy = fy - iy.astype(jnp.float32)
            wz = fz - iz.astype(jnp.float32)
            u = (1.0 - wx, wx)
            v = (1.0 - wy, wy)
            t = (1.0 - wz, wz)
            pxy = [u[i] * v[j] for i in (0, 1) for j in (0, 1)]
            return pxy, t

        def chunk_body(ch, carry):
            base = wid * npt + ch * B
            pltpu.sync_copy(xx_h.at[pl.ds(base, B)], xbx)
            pltpu.sync_copy(xy_h.at[pl.ds(base, B)], xby)
            pltpu.sync_copy(xz_h.at[pl.ds(base, B)], xbz)

            def idx_loop(l):
                lbase = l * T
                bi = l % 2

                def body(g, c, l=l, lbase=lbase, bi=bi):
                    _, _, _, ix, iy, iz = coords(g, l)
                    rows = corner_rows(l, ix, iy, iz)
                    for ci in range(8):
                        idxb[bi][pl.ds(ci * B + g * 16, 16)] = (rows[ci] + lbase) >> 2
                    return c

                lax.fori_loop(0, GROUPS, body, 0)

            def fire(l):
                bi = l % 2
                return pltpu.async_copy(tw8_h.at[idxb[bi]], dstb[bi], sems[bi])

            def acc_loop(l):
                bi = l % 2

                def body(g, c, l=l, bi=bi):
                    fx, fy, fz, ix, iy, iz = coords(g, l)
                    rows = corner_rows(l, ix, iy, iz)
                    pxy, t = weights(fx, fy, fz, ix, iy, iz)
                    prow = g * 16 + iota16
                    acc0 = acc1 = None
                    for ci, (i, j, k) in enumerate(CORNERS):
                        wc = pxy[2 * i + j] * t[k]
                        ocol = (rows[ci] & 3) * 2
                        v0 = plsc.load_gather(dstb[bi], [prow + ci * B, ocol])
                        v1 = plsc.load_gather(dstb[bi], [prow + ci * B, ocol + 1])
                        if acc0 is None:
                            acc0, acc1 = wc * v0, wc * v1
                        else:
                            acc0, acc1 = acc0 + wc * v0, acc1 + wc * v1
                    peb[2 * l, pl.ds(g * 16, 16)] = acc0
                    peb[2 * l + 1, pl.ds(g * 16, 16)] = acc1
                    return c

                lax.fori_loop(0, GROUPS, body, 0)

            def resident_loop(l, tab):
                def body(g, c, l=l):
                    fx, fy, fz, ix, iy, iz = coords(g, l)
                    rows = corner_rows(l, ix, iy, iz)
                    pxy, t = weights(fx, fy, fz, ix, iy, iz)
                    acc0 = acc1 = None
                    for ci, (i, j, k) in enumerate(CORNERS):
                        wc = pxy[2 * i + j] * t[k]
                        r8 = rows[ci] >> 2
                        oc = (rows[ci] & 3) * 2
                        v0 = plsc.load_gather(tab, [r8, oc])
                        v1 = plsc.load_gather(tab, [r8, oc + 1])
                        if acc0 is None:
                            acc0, acc1 = wc * v0, wc * v1
                        else:
                            acc0, acc1 = acc0 + wc * v0, acc1 + wc * v1
                    peb[2 * l, pl.ds(g * 16, 16)] = acc0
                    peb[2 * l + 1, pl.ds(g * 16, 16)] = acc1
                    return c

                lax.fori_loop(0, GROUPS, body, 0)

            idx_loop(2)
            d_prev = fire(2)
            resident_loop(0, tab0)
            resident_loop(1, tab1)
            for l in range(3, N_LEVELS):
                idx_loop(l)
                d_next = fire(l)
                d_prev.wait()
                acc_loop(l - 1)
                d_prev = d_next
            d_prev.wait()
            acc_loop(N_LEVELS - 1)

            pltpu.sync_copy(peb, pe_h.at[:, pl.ds(base, B)])
            return carry

        lax.fori_loop(0, nch, chunk_body, 0)

    return enc(xx, xy, xz, tw2.reshape(-1, 8))


def _mlp(pe, W0, W1, W2):
    N = pe.shape[0]
    BN = 1024
    dn = (((1,), (0,)), ((), ()))

    def body(pe_ref, w0_ref, w1_ref, w2_ref, z_ref, d_ref):
        p = pe_ref[...]
        h = jnp.maximum(
            lax.dot_general(p, w0_ref[...], dn, preferred_element_type=jnp.float32), 0.0)
        h = jnp.maximum(
            lax.dot_general(h, w1_ref[...], dn, preferred_element_type=jnp.float32), 0.0)
        z = lax.dot_general(h, w2_ref[...], dn, preferred_element_type=jnp.float32)
        z_ref[...] = z
        z0 = z[:, 0:1]
        d_ref[...] = jnp.maximum(z0, 0.0) + jnp.log1p(jnp.exp(-jnp.abs(z0)))

    z, dens = pl.pallas_call(
        body,
        grid=(N // BN,),
        in_specs=[
            pl.BlockSpec((BN, 2 * N_LEVELS), lambda i: (i, 0)),
            pl.BlockSpec((2 * N_LEVELS, 64), lambda i: (0, 0)),
            pl.BlockSpec((64, 64), lambda i: (0, 0)),
            pl.BlockSpec((64, 16), lambda i: (0, 0)),
        ],
        out_specs=[
            pl.BlockSpec((BN, 16), lambda i: (i, 0)),
            pl.BlockSpec((BN, 1), lambda i: (i, 0)),
        ],
        out_shape=[
            jax.ShapeDtypeStruct((N, 16), jnp.float32),
            jax.ShapeDtypeStruct((N, 1), jnp.float32),
        ],
    )(pe, W0, W1, W2)
    return z, dens


def kernel(x, tables, W0, W1, W2):
    N = x.shape[0]
    xn = x * jnp.float32(1.0 / 256.0)
    xnt = xn.T
    pe = _encode(xnt[0], xnt[1], xnt[2], tables.reshape(N_LEVELS * T, F)).T
    z, dens = _mlp(pe, W0, W1, W2)
    return (dens.reshape(x.shape[:-1]), pe, z)
